# Initial kernel scaffold; baseline (speedup 1.0000x reference)
import jax
import jax.numpy as jnp
from jax.experimental import pallas as pl


def kernel(target, context, target_table, context_table):
    raise NotImplementedError("write your pallas kernel here")


# speedup vs baseline: 1.4197x; 1.4197x over previous
import functools

import jax
import jax.numpy as jnp
from jax import lax
from jax.experimental import pallas as pl
from jax.experimental.pallas import tpu as pltpu
from jax.experimental.pallas import tpu_sc as plsc

VOCAB = 100002
DIM = 128
BATCH = 16384
C = 5
NC = 2
NS = 16
L = 16
NW = NC * NS
B_PER_W = BATCH // NW
CB = 16
N_CHUNKS = B_PER_W // CB


def _dots_body(tgt_hbm, ctx_hbm, ttab_hbm, ctab_hbm, out_hbm,
               idx_t, idx_c, rows_t, rows_c, out_v, sem_t, sem_c):
    wid = lax.axis_index("s") * NC + lax.axis_index("c")
    lane = lax.iota(jnp.int32, L)

    def chunk(ch, _):
        base = wid * B_PER_W + ch * CB
        pltpu.sync_copy(tgt_hbm.at[pl.ds(base, CB)], idx_t)
        pltpu.sync_copy(ctx_hbm.at[pl.ds(base * C, CB * C)], idx_c)
        cp_t = pltpu.async_copy(ttab_hbm.at[idx_t], rows_t, sem_t)
        cp_c = pltpu.async_copy(ctab_hbm.at[idx_c], rows_c, sem_c)
        cp_t.wait()
        cp_c.wait()

        def e_step(e, accs):
            a0, a1, a2, a3, a4 = accs
            col = jnp.full((L,), e, jnp.int32)
            w = plsc.load_gather(rows_t, [lane, col])
            r5 = lane * C
            a0 = a0 + w * plsc.load_gather(rows_c, [r5, col])
            a1 = a1 + w * plsc.load_gather(rows_c, [r5 + 1, col])
            a2 = a2 + w * plsc.load_gather(rows_c, [r5 + 2, col])
            a3 = a3 + w * plsc.load_gather(rows_c, [r5 + 3, col])
            a4 = a4 + w * plsc.load_gather(rows_c, [r5 + 4, col])
            return a0, a1, a2, a3, a4

        z = jnp.zeros((L,), jnp.float32)
        accs = lax.fori_loop(0, DIM, e_step, (z, z, z, z, z))
        r5 = lane * C
        for c in range(C):
            plsc.store_scatter(out_v, [r5 + c], accs[c])
        pltpu.sync_copy(out_v, out_hbm.at[pl.ds(base * C, CB * C)])
        return ()

    lax.fori_loop(0, N_CHUNKS, chunk, ())


@jax.jit
def _dots(target_flat, context_flat, target_table, context_table):
    mesh = plsc.VectorSubcoreMesh(
        core_axis_name="c", subcore_axis_name="s",
        num_cores=NC, num_subcores=NS)
    return pl.kernel(
        _dots_body,
        out_type=jax.ShapeDtypeStruct((BATCH * C,), jnp.float32),
        mesh=mesh,
        compiler_params=pltpu.CompilerParams(needs_layout_passes=False),
        scratch_types=[
            pltpu.VMEM((CB,), jnp.int32),
            pltpu.VMEM((CB * C,), jnp.int32),
            pltpu.VMEM((CB, DIM), jnp.float32),
            pltpu.VMEM((CB * C, DIM), jnp.float32),
            pltpu.VMEM((CB * C,), jnp.float32),
            pltpu.SemaphoreType.DMA,
            pltpu.SemaphoreType.DMA,
        ],
    )(target_flat, context_flat, target_table, context_table)


def kernel(target, context, target_table, context_table):
    target_flat = target.reshape(-1).astype(jnp.int32)
    context_flat = context.reshape(-1).astype(jnp.int32)
    out = _dots(target_flat, context_flat, target_table, context_table)
    return out.reshape(BATCH, C)


# speedup vs baseline: 2.0846x; 1.4683x over previous
import jax
import jax.numpy as jnp
from jax import lax
from jax.experimental import pallas as pl
from jax.experimental.pallas import tpu as pltpu
from jax.experimental.pallas import tpu_sc as plsc

VOCAB = 100002
DIM = 128
BATCH = 16384
C = 5
NC = 2
NS = 16
L = 16
NW = NC * NS
B_PER_W = BATCH // NW
CB = 16
N_CHUNKS = B_PER_W // CB


def _dots_body(tgt_hbm, ctx_hbm, ttab_hbm, ctab_hbm, out_hbm,
               idx_t, idx_c, rows_t2, rows_c2, out_v,
               sem_t0, sem_t1, sem_c0, sem_c1):
    wid = lax.axis_index("s") * NC + lax.axis_index("c")
    lane = lax.iota(jnp.int32, L)
    r5 = lane * C

    pltpu.sync_copy(tgt_hbm.at[pl.ds(wid * B_PER_W, B_PER_W)], idx_t)
    pltpu.sync_copy(ctx_hbm.at[pl.ds(wid * B_PER_W * C, B_PER_W * C)], idx_c)

    sem_t = (sem_t0, sem_t1)
    sem_c = (sem_c0, sem_c1)

    def gather_descs(g, b):
        ot = pl.multiple_of(g * CB, 8)
        oc = pl.multiple_of(g * (CB * C), 8)
        dt = pltpu.make_async_copy(
            ttab_hbm.at[idx_t.at[pl.ds(ot, CB)]], rows_t2.at[b], sem_t[b])
        dc = pltpu.make_async_copy(
            ctab_hbm.at[idx_c.at[pl.ds(oc, CB * C)]], rows_c2.at[b], sem_c[b])
        return dt, dc

    def issue(g, b):
        dt, dc = gather_descs(g, b)
        dt.start()
        dc.start()

    issue(0, 0)
    issue(1, 1)

    @pl.loop(0, N_CHUNKS, step=2)
    def _chunks(ch):
        for b in range(2):
            g = ch + b
            dt, dc = gather_descs(g, b)
            dt.wait()
            dc.wait()
            rows_t = rows_t2.at[b]
            rows_c = rows_c2.at[b]

            @pl.loop(0, DIM, init_carry=tuple(jnp.zeros((L,), jnp.float32)
                                              for _ in range(C)), unroll=8)
            def accs(e, carry):
                a0, a1, a2, a3, a4 = carry
                col = jnp.full((L,), e, jnp.int32)
                w = plsc.load_gather(rows_t, [lane, col])
                a0 = a0 + w * plsc.load_gather(rows_c, [r5, col])
                a1 = a1 + w * plsc.load_gather(rows_c, [r5 + 1, col])
                a2 = a2 + w * plsc.load_gather(rows_c, [r5 + 2, col])
                a3 = a3 + w * plsc.load_gather(rows_c, [r5 + 3, col])
                a4 = a4 + w * plsc.load_gather(rows_c, [r5 + 4, col])
                return a0, a1, a2, a3, a4

            obase = g * (CB * C) + r5
            for c in range(C):
                plsc.store_scatter(out_v, [obase + c], accs[c])

            @pl.when(g + 2 < N_CHUNKS)
            def _prefetch():
                issue(g + 2, b)

    pltpu.sync_copy(out_v, out_hbm.at[pl.ds(wid * (B_PER_W * C),
                                            B_PER_W * C)])


@jax.jit
def _dots(target_flat, context_flat, target_table, context_table):
    mesh = plsc.VectorSubcoreMesh(
        core_axis_name="c", subcore_axis_name="s",
        num_cores=NC, num_subcores=NS)
    return pl.kernel(
        _dots_body,
        out_type=jax.ShapeDtypeStruct((BATCH * C,), jnp.float32),
        mesh=mesh,
        compiler_params=pltpu.CompilerParams(needs_layout_passes=False),
        scratch_types=[
            pltpu.VMEM((B_PER_W,), jnp.int32),
            pltpu.VMEM((B_PER_W * C,), jnp.int32),
            pltpu.VMEM((2, CB, DIM), jnp.float32),
            pltpu.VMEM((2, CB * C, DIM), jnp.float32),
            pltpu.VMEM((B_PER_W * C,), jnp.float32),
            pltpu.SemaphoreType.DMA,
            pltpu.SemaphoreType.DMA,
            pltpu.SemaphoreType.DMA,
            pltpu.SemaphoreType.DMA,
        ],
    )(target_flat, context_flat, target_table, context_table)


def kernel(target, context, target_table, context_table):
    target_flat = target.reshape(-1).astype(jnp.int32)
    context_flat = context.reshape(-1).astype(jnp.int32)
    out = _dots(target_flat, context_flat, target_table, context_table)
    return out.reshape(BATCH, C)


# speedup vs baseline: 5.9232x; 2.8414x over previous
import jax
import jax.numpy as jnp
from jax import lax
from jax.experimental import pallas as pl
from jax.experimental.pallas import tpu as pltpu
from jax.experimental.pallas import tpu_sc as plsc

VOCAB = 100002
DIM = 128
BATCH = 16384
C = 5
NC = 2
NS = 16
L = 16
NW = NC * NS
B_PER_W = BATCH // NW
CB = 16
N_CHUNKS = B_PER_W // CB


def _dots_body(tgt_hbm, ctx_hbm, ttab_hbm, ctab_hbm, out_hbm,
               idx_t, idx_c, rows_t2, rows_c2, out_v,
               sem_t0, sem_t1, sem_c0, sem_c1):
    wid = lax.axis_index("s") * NC + lax.axis_index("c")
    lane = lax.iota(jnp.int32, L)
    r5 = lane * C

    pltpu.sync_copy(tgt_hbm.at[pl.ds(wid * B_PER_W, B_PER_W)], idx_t)
    pltpu.sync_copy(ctx_hbm.at[pl.ds(wid * B_PER_W * C, B_PER_W * C)], idx_c)

    sem_t = (sem_t0, sem_t1)
    sem_c = (sem_c0, sem_c1)

    def gather_descs(g, b):
        ot = pl.multiple_of(g * CB, 8)
        oc = pl.multiple_of(g * (CB * C), 8)
        dt = pltpu.make_async_copy(
            ttab_hbm.at[idx_t.at[pl.ds(ot, CB)]], rows_t2.at[b], sem_t[b])
        dc = pltpu.make_async_copy(
            ctab_hbm.at[idx_c.at[pl.ds(oc, CB * C)]], rows_c2.at[b], sem_c[b])
        return dt, dc

    def issue(g, b):
        dt, dc = gather_descs(g, b)
        dt.start()
        dc.start()

    issue(0, 0)
    issue(1, 1)

    @pl.loop(0, N_CHUNKS, step=2)
    def _chunks(ch):
        for b in range(2):
            g = ch + b
            dt, dc = gather_descs(g, b)
            dt.wait()
            dc.wait()
            rows_t = rows_t2.at[b]
            rows_c = rows_c2.at[b]

            @pl.loop(0, DIM, init_carry=tuple(jnp.zeros((L,), jnp.float32)
                                              for _ in range(C)), unroll=8)
            def accs(e, carry):
                a0, a1, a2, a3, a4 = carry
                col = (lane + e) & (DIM - 1)
                w = plsc.load_gather(rows_t, [lane, col])
                a0 = a0 + w * plsc.load_gather(rows_c, [r5, col])
                a1 = a1 + w * plsc.load_gather(rows_c, [r5 + 1, col])
                a2 = a2 + w * plsc.load_gather(rows_c, [r5 + 2, col])
                a3 = a3 + w * plsc.load_gather(rows_c, [r5 + 3, col])
                a4 = a4 + w * plsc.load_gather(rows_c, [r5 + 4, col])
                return a0, a1, a2, a3, a4

            obase = g * (CB * C) + r5
            for c in range(C):
                plsc.store_scatter(out_v, [obase + c], accs[c])

            @pl.when(g + 2 < N_CHUNKS)
            def _prefetch():
                issue(g + 2, b)

    pltpu.sync_copy(out_v, out_hbm.at[pl.ds(wid * (B_PER_W * C),
                                            B_PER_W * C)])


@jax.jit
def _dots(target_flat, context_flat, target_table, context_table):
    mesh = plsc.VectorSubcoreMesh(
        core_axis_name="c", subcore_axis_name="s",
        num_cores=NC, num_subcores=NS)
    return pl.kernel(
        _dots_body,
        out_type=jax.ShapeDtypeStruct((BATCH * C,), jnp.float32),
        mesh=mesh,
        compiler_params=pltpu.CompilerParams(needs_layout_passes=False),
        scratch_types=[
            pltpu.VMEM((B_PER_W,), jnp.int32),
            pltpu.VMEM((B_PER_W * C,), jnp.int32),
            pltpu.VMEM((2, CB, DIM), jnp.float32),
            pltpu.VMEM((2, CB * C, DIM), jnp.float32),
            pltpu.VMEM((B_PER_W * C,), jnp.float32),
            pltpu.SemaphoreType.DMA,
            pltpu.SemaphoreType.DMA,
            pltpu.SemaphoreType.DMA,
            pltpu.SemaphoreType.DMA,
        ],
    )(target_flat, context_flat, target_table, context_table)


def kernel(target, context, target_table, context_table):
    target_flat = target.reshape(-1).astype(jnp.int32)
    context_flat = context.reshape(-1).astype(jnp.int32)
    out = _dots(target_flat, context_flat, target_table, context_table)
    return out.reshape(BATCH, C)


# speedup vs baseline: 6.2732x; 1.0591x over previous
import jax
import jax.numpy as jnp
from jax import lax
from jax.experimental import pallas as pl
from jax.experimental.pallas import tpu as pltpu
from jax.experimental.pallas import tpu_sc as plsc

VOCAB = 100002
DIM = 128
BATCH = 16384
C = 5
NC = 2
NS = 16
L = 16
NW = NC * NS
B_PER_W = BATCH // NW
CB = 32
N_CHUNKS = B_PER_W // CB
NG = CB // L


def _dots_body(tgt_hbm, ctx_hbm, ttab_hbm, ctab_hbm, out_hbm,
               idx_t, idx_c, rows_t2, rows_c2, out_v,
               sem_t0, sem_t1, sem_c0, sem_c1):
    wid = lax.axis_index("s") * NC + lax.axis_index("c")
    lane = lax.iota(jnp.int32, L)
    r5 = lane * C

    pltpu.sync_copy(tgt_hbm.at[pl.ds(wid * B_PER_W, B_PER_W)], idx_t)
    pltpu.sync_copy(ctx_hbm.at[pl.ds(wid * B_PER_W * C, B_PER_W * C)], idx_c)

    sem_t = (sem_t0, sem_t1)
    sem_c = (sem_c0, sem_c1)
    HALF = CB * C // 2

    def gather_descs(g, b):
        ot = pl.multiple_of(g * CB, 8)
        oc = pl.multiple_of(g * (CB * C), 8)
        dt = pltpu.make_async_copy(
            ttab_hbm.at[idx_t.at[pl.ds(ot, CB)]], rows_t2.at[b], sem_t[b])
        dc0 = pltpu.make_async_copy(
            ctab_hbm.at[idx_c.at[pl.ds(oc, HALF)]],
            rows_c2.at[b, pl.ds(0, HALF)], sem_c[b])
        dc1 = pltpu.make_async_copy(
            ctab_hbm.at[idx_c.at[pl.ds(oc + HALF, HALF)]],
            rows_c2.at[b, pl.ds(HALF, HALF)], sem_c[b])
        return dt, dc0, dc1

    def issue(g, b):
        for d in gather_descs(g, b):
            d.start()

    issue(0, 0)
    issue(1, 1)

    @pl.loop(0, N_CHUNKS, step=2)
    def _chunks(ch):
        for b in range(2):
            g = ch + b
            dt, dc0, dc1 = gather_descs(g, b)
            dt.wait()
            dc0.wait()
            dc1.wait()

            for q in range(NG):
                rows_t = rows_t2.at[b, pl.ds(q * L, L)]
                rows_c = rows_c2.at[b, pl.ds(q * L * C, L * C)]

                @pl.loop(0, DIM,
                         init_carry=tuple(jnp.zeros((L,), jnp.float32)
                                          for _ in range(C)), unroll=8)
                def accs(e, carry):
                    a0, a1, a2, a3, a4 = carry
                    col = (lane + e) & (DIM - 1)
                    w = plsc.load_gather(rows_t, [lane, col])
                    a0 = a0 + w * plsc.load_gather(rows_c, [r5, col])
                    a1 = a1 + w * plsc.load_gather(rows_c, [r5 + 1, col])
                    a2 = a2 + w * plsc.load_gather(rows_c, [r5 + 2, col])
                    a3 = a3 + w * plsc.load_gather(rows_c, [r5 + 3, col])
                    a4 = a4 + w * plsc.load_gather(rows_c, [r5 + 4, col])
                    return a0, a1, a2, a3, a4

                obase = g * (CB * C) + q * (L * C) + r5
                for c in range(C):
                    plsc.store_scatter(out_v, [obase + c], accs[c])

            @pl.when(g + 2 < N_CHUNKS)
            def _prefetch():
                issue(g + 2, b)

    pltpu.sync_copy(out_v, out_hbm.at[pl.ds(wid * (B_PER_W * C),
                                            B_PER_W * C)])


@jax.jit
def _dots(target_flat, context_flat, target_table, context_table):
    mesh = plsc.VectorSubcoreMesh(
        core_axis_name="c", subcore_axis_name="s",
        num_cores=NC, num_subcores=NS)
    return pl.kernel(
        _dots_body,
        out_type=jax.ShapeDtypeStruct((BATCH * C,), jnp.float32),
        mesh=mesh,
        compiler_params=pltpu.CompilerParams(needs_layout_passes=False),
        scratch_types=[
            pltpu.VMEM((B_PER_W,), jnp.int32),
            pltpu.VMEM((B_PER_W * C,), jnp.int32),
            pltpu.VMEM((2, CB, DIM), jnp.float32),
            pltpu.VMEM((2, CB * C, DIM), jnp.float32),
            pltpu.VMEM((B_PER_W * C,), jnp.float32),
            pltpu.SemaphoreType.DMA,
            pltpu.SemaphoreType.DMA,
            pltpu.SemaphoreType.DMA,
            pltpu.SemaphoreType.DMA,
        ],
    )(target_flat, context_flat, target_table, context_table)


def kernel(target, context, target_table, context_table):
    target_flat = target.reshape(-1).astype(jnp.int32)
    context_flat = context.reshape(-1).astype(jnp.int32)
    out = _dots(target_flat, context_flat, target_table, context_table)
    return out.reshape(BATCH, C)
